# Initial kernel scaffold; baseline (speedup 1.0000x reference)
import jax
import jax.numpy as jnp
from jax.experimental import pallas as pl


def kernel(x, edge_index, batch, Re, bafflesze, dbl, W_down0, b_down0, W_down1, b_down1, p_pool0, p_pool1, W_bot1, b_bot1, W_bot2, b_bot2, W_up0, b_up0, W_up1, b_up1):
    raise NotImplementedError("write your pallas kernel here")


# speedup vs baseline: 10.9085x; 10.9085x over previous
import functools

import jax
import jax.numpy as jnp
from jax import lax
from jax.experimental import pallas as pl
from jax.experimental.pallas import tpu as pltpu
from jax.experimental.pallas import tpu_sc as plsc

N = 10000
E = 320000
F = 128
LAT = 128
NC = 8
RS = 16
K1 = 5000
K2 = 2500

N_PAD = 10240
BR = 512
GRID = N_PAD // BR
C = 128
NW = 32
NCH = 79
E_PAD = NW * NCH * C


def _make_agg(f):
    rpt = N_PAD // 16
    mesh = plsc.VectorSubcoreMesh(core_axis_name="c", subcore_axis_name="s")

    @functools.partial(
        pl.kernel,
        out_type=(
            jax.ShapeDtypeStruct((2, N_PAD, f), jnp.float32),
            jax.ShapeDtypeStruct((2, N_PAD), jnp.float32),
        ),
        mesh=mesh,
        scratch_types=[
            pltpu.VMEM((NCH, C), jnp.int32),
            pltpu.VMEM((NCH, C), jnp.int32),
            pltpu.VMEM((C, f), jnp.float32),
            pltpu.VMEM((C,), jnp.float32),
            pltpu.VMEM_SHARED((N_PAD, f), jnp.float32),
            pltpu.VMEM_SHARED((N_PAD,), jnp.float32),
            pltpu.SemaphoreType.DMA,
            pltpu.SemaphoreType.DMA,
        ],
        compiler_params=pltpu.CompilerParams(use_tc_tiling_on_sc=False),
    )
    def agg(table, wtab, srcm, dstm, ztab, zcol,
            acc_out, deg_out, srcv, dstv, gbuf, wbuf, accs, degs, sem1, sem2):
        c = lax.axis_index("c")
        s = lax.axis_index("s")
        wid = s * 2 + c
        r0 = s * rpt
        pltpu.sync_copy(ztab.at[pl.ds(r0, rpt)], accs.at[pl.ds(r0, rpt)])
        pltpu.sync_copy(zcol.at[pl.ds(r0, rpt)], degs.at[pl.ds(r0, rpt)])
        pltpu.sync_copy(srcm.at[wid], srcv)
        pltpu.sync_copy(dstm.at[wid], dstv)
        plsc.subcore_barrier()

        def body(j, carry):
            pltpu.async_copy(table.at[srcv.at[j]], gbuf, sem1).wait()
            pltpu.async_copy(wtab.at[srcv.at[j]], wbuf, sem2).wait()
            pltpu.sync_copy(gbuf, accs.at[dstv.at[j]], add=True)
            pltpu.sync_copy(wbuf, degs.at[dstv.at[j]], add=True)
            return carry

        lax.fori_loop(0, NCH, body, 0)
        plsc.subcore_barrier()
        pltpu.sync_copy(accs.at[pl.ds(r0, rpt)], acc_out.at[c, pl.ds(r0, rpt)])
        pltpu.sync_copy(degs.at[pl.ds(r0, rpt)], deg_out.at[c, pl.ds(r0, rpt)])

    return agg


_agg_f = _make_agg(F)
_agg_nc = _make_agg(NC)


def _mm_body(x_ref, w_ref, o_ref):
    o_ref[...] = jnp.dot(x_ref[...], w_ref[...],
                         preferred_element_type=jnp.float32)


def _conv_finish_body(y_ref, acc_ref, deg_ref, b_ref, p_ref, h_ref, s_ref):
    acc = acc_ref[0] + acc_ref[1]
    deg = deg_ref[0, :, 0] + deg_ref[1, :, 0]
    h = jax.nn.relu(y_ref[...] + acc / jnp.maximum(deg, 1.0)[:, None]
                    + b_ref[...])
    h_ref[...] = h
    p = p_ref[...]
    s_ref[...] = jnp.sum(h * p[None, :], axis=1) / jnp.sqrt(jnp.sum(p * p))


def _pool_table_body(h_ref, s_ref, kept_ref, w_ref, o_ref):
    t = jnp.tanh(s_ref[...])
    o_ref[...] = (jnp.dot(h_ref[...] * t[:, None], w_ref[...],
                          preferred_element_type=jnp.float32)
                  * kept_ref[...][:, None])


def _bottleneck_body(g1_ref, s1_ref, k1_ref, k0_ref, wb1_ref, bb1_ref,
                     wb2_ref, bb2_ref, wu0_ref, re_ref, baf_ref, dbl_ref,
                     o_ref):
    g1 = g1_ref[...]
    g = g1 * jnp.tanh(s1_ref[...])[:, None]
    w = wb1_ref[...]
    crow = (re_ref[0] * jnp.sum(w[LAT:LAT + RS], axis=0)
            + baf_ref[0] * jnp.sum(w[LAT + RS:LAT + 2 * RS], axis=0)
            + dbl_ref[0] * jnp.sum(w[LAT + 2 * RS:], axis=0)) + bb1_ref[...]
    hh = jax.nn.relu(jnp.dot(g, w[:LAT],
                             preferred_element_type=jnp.float32) + crow)
    h2 = jax.nn.relu(jnp.dot(hh, wb2_ref[...],
                             preferred_element_type=jnp.float32) + bb2_ref[...])
    wu = wu0_ref[...]
    o_ref[...] = ((jnp.dot(k1_ref[...][:, None] * h2, wu[:LAT],
                           preferred_element_type=jnp.float32)
                   + jnp.dot(g1, wu[LAT:],
                             preferred_element_type=jnp.float32))
                  * k0_ref[...][:, None])


def _up0_finish_body(y2_ref, acc_ref, deg_ref, b_ref, k0_ref, h1_ref,
                     wu1_ref, o_ref):
    acc = acc_ref[0] + acc_ref[1]
    deg = deg_ref[0, :, 0] + deg_ref[1, :, 0]
    g2 = jax.nn.relu(y2_ref[...] + acc / jnp.maximum(deg, 1.0)[:, None]
                     + b_ref[...])
    wu = wu1_ref[...]
    o_ref[...] = (jnp.dot(k0_ref[...][:, None] * g2, wu[:LAT],
                          preferred_element_type=jnp.float32)
                  + jnp.dot(h1_ref[...], wu[LAT:],
                            preferred_element_type=jnp.float32))


def _final_body(y3_ref, acc_ref, deg_ref, b_ref, o_ref):
    acc = acc_ref[0] + acc_ref[1]
    deg = deg_ref[0, :, 0] + deg_ref[1, :, 0]
    r = jax.nn.relu(y3_ref[...] + acc / jnp.maximum(deg, 1.0)[:, None]
                    + b_ref[...])
    m = jnp.max(r, axis=1, keepdims=True)
    lse = jnp.log(jnp.sum(jnp.exp(r - m), axis=1, keepdims=True)) + m
    o_ref[...] = r - lse


def _topk_body(k, s_ref, valid_ref, o_ref):
    s = s_ref[...]
    ub = lax.bitcast_convert_type(s, jnp.uint32)
    sign = ub >> jnp.uint32(31)
    ukey = jnp.where(sign == jnp.uint32(1), ~ub,
                     ub | jnp.uint32(0x80000000))
    ukey = jnp.where(valid_ref[...] > 0.0, ukey, jnp.uint32(0))

    def tbit(i, t):
        cand = t | (jnp.uint32(1) << (jnp.uint32(31) - i.astype(jnp.uint32)))
        cnt = jnp.sum(jnp.where(ukey >= cand, 1, 0).astype(jnp.int32))
        return jnp.where(cnt >= k, cand, t)

    thr = lax.fori_loop(0, 32, tbit, jnp.uint32(0))
    above = ukey > thr
    need = k - jnp.sum(jnp.where(above, 1, 0).astype(jnp.int32))
    tie = ukey == thr
    idx = lax.broadcasted_iota(jnp.int32, (N_PAD,), 0)

    def mbit(i, t):
        cand = t | (1 << (13 - i))
        cnt = jnp.sum(jnp.where(tie & (idx < cand), 1, 0).astype(jnp.int32))
        return jnp.where(cnt < need, cand, t)

    mhat = lax.fori_loop(0, 14, mbit, jnp.int32(0))
    mstar = jnp.where(need > 0, mhat + 1, 0)
    o_ref[...] = (above | (tie & (idx < mstar))).astype(jnp.float32)


def _row_spec(f):
    return pl.BlockSpec((BR, f), lambda i: (i, 0))


def _full_spec(shape):
    nd = len(shape)
    return pl.BlockSpec(shape, lambda i: (0,) * nd)


_VEC = pl.BlockSpec((BR,), lambda i: (i,))
_ACC = pl.BlockSpec((2, BR, F), lambda i: (0, i, 0))
_ACC8 = pl.BlockSpec((2, BR, NC), lambda i: (0, i, 0))
_DEG = pl.BlockSpec((2, BR, 1), lambda i: (0, i, 0))
_SMEM = pl.BlockSpec(memory_space=pltpu.SMEM)


def _call(body, in_specs, out_specs, out_shape, args, grid=(GRID,)):
    return pl.pallas_call(
        body, grid=grid, in_specs=in_specs, out_specs=out_specs,
        out_shape=out_shape)(*args)


def kernel(x, edge_index, batch, Re, bafflesze, dbl,
           W_down0, b_down0, W_down1, b_down1, p_pool0, p_pool1,
           W_bot1, b_bot1, W_bot2, b_bot2, W_up0, b_up0, W_up1, b_up1):
    src = edge_index[0].astype(jnp.int32)
    dst = edge_index[1].astype(jnp.int32)
    padn = jnp.full((E_PAD - E,), N, jnp.int32)
    srcm = jnp.concatenate([src, padn]).reshape(NW, NCH, C)
    dstm = jnp.concatenate([dst, padn]).reshape(NW, NCH, C)
    x_pad = jnp.pad(x, ((0, N_PAD - N), (0, 0)))
    row_valid = (jnp.arange(N_PAD) < N).astype(jnp.float32)
    ztab = jnp.zeros((N_PAD, F), jnp.float32)
    ztab8 = jnp.zeros((N_PAD, NC), jnp.float32)
    zcol = jnp.zeros((N_PAD,), jnp.float32)

    f32 = jnp.float32
    sds = jax.ShapeDtypeStruct

    y0 = _call(_mm_body, [_row_spec(F), _full_spec((F, F))], _row_spec(F),
               sds((N_PAD, F), f32), (x_pad, W_down0))
    acc0, deg0 = _agg_f(y0, row_valid, srcm, dstm, ztab, zcol)
    h1, s0 = _call(
        _conv_finish_body,
        [_row_spec(F), _ACC, _DEG, _full_spec((F,)), _full_spec((F,))],
        (_row_spec(F), _VEC),
        (sds((N_PAD, F), f32), sds((N_PAD,), f32)),
        (y0, acc0, deg0.reshape(2, N_PAD, 1), b_down0, p_pool0))

    kept0 = _call(functools.partial(_topk_body, K1),
                  [_full_spec((N_PAD,)), _full_spec((N_PAD,))],
                  _full_spec((N_PAD,)), sds((N_PAD,), f32), (s0, row_valid),
                  grid=(1,))

    y1k = _call(_pool_table_body,
                [_row_spec(F), _VEC, _VEC, _full_spec((F, F))], _row_spec(F),
                sds((N_PAD, F), f32), (h1, s0, kept0, W_down1))
    acc1, deg1 = _agg_f(y1k, kept0, srcm, dstm, ztab, zcol)
    g1, s1 = _call(
        _conv_finish_body,
        [_row_spec(F), _ACC, _DEG, _full_spec((F,)), _full_spec((F,))],
        (_row_spec(F), _VEC),
        (sds((N_PAD, F), f32), sds((N_PAD,), f32)),
        (y1k, acc1, deg1.reshape(2, N_PAD, 1), b_down1, p_pool1))

    kept1 = _call(functools.partial(_topk_body, K2),
                  [_full_spec((N_PAD,)), _full_spec((N_PAD,))],
                  _full_spec((N_PAD,)), sds((N_PAD,), f32), (s1, kept0),
                  grid=(1,))

    y2k = _call(
        _bottleneck_body,
        [_row_spec(F), _VEC, _VEC, _VEC, _full_spec((LAT + 3 * RS, 88)),
         _full_spec((88,)), _full_spec((88, LAT)), _full_spec((LAT,)),
         _full_spec((2 * LAT, F)), _SMEM, _SMEM, _SMEM],
        _row_spec(F), sds((N_PAD, F), f32),
        (g1, s1, kept1, kept0, W_bot1, b_bot1, W_bot2, b_bot2, W_up0,
         Re, bafflesze, dbl))
    acc2, deg2 = _agg_f(y2k, kept0, srcm, dstm, ztab, zcol)
    y3 = _call(
        _up0_finish_body,
        [_row_spec(F), _ACC, _DEG, _full_spec((F,)), _VEC, _row_spec(F),
         _full_spec((2 * LAT, NC))],
        _row_spec(NC), sds((N_PAD, NC), f32),
        (y2k, acc2, deg2.reshape(2, N_PAD, 1), b_up0, kept0, h1, W_up1))

    acc3, deg3 = _agg_nc(y3, row_valid, srcm, dstm, ztab8, zcol)
    out = _call(
        _final_body,
        [_row_spec(NC), _ACC8, _DEG, _full_spec((NC,))],
        _row_spec(NC), sds((N_PAD, NC), f32),
        (y3, acc3, deg3.reshape(2, N_PAD, 1), b_up1))
    return out[:N]
